# baseline (speedup 1.0000x reference)
import functools

import jax
import jax.numpy as jnp
from jax import lax
from jax.experimental import pallas as pl
from jax.experimental.pallas import tpu as pltpu
from jax.experimental.pallas import tpu_sc as plsc

NUM_NOTES = 91
EMB = 8
OUT_D = 10
LANES = 16
NW = 32
NBUF = 3
C = 1024


@functools.lru_cache(maxsize=None)
def _build(N):
    n_per_w = N // NW
    n_chunks = n_per_w // C
    n_tiles = N // C

    mesh = plsc.VectorSubcoreMesh(core_axis_name="c", subcore_axis_name="s")

    @functools.partial(
        pl.kernel,
        mesh=mesh,
        out_type=jax.ShapeDtypeStruct((OUT_D, n_tiles, 8, 128), jnp.float32),
        scratch_types=[
            pltpu.VMEM((NUM_NOTES * EMB,), jnp.float32),
            [pltpu.VMEM((C,), jnp.int32) for _ in range(NBUF)],
            [pltpu.VMEM((OUT_D, 8, 128), jnp.float32) for _ in range(NBUF)],
            [pltpu.SemaphoreType.DMA for _ in range(NBUF)],
            [pltpu.SemaphoreType.DMA for _ in range(NBUF)],
        ],
        compiler_params=pltpu.CompilerParams(
            needs_layout_passes=False, use_tc_tiling_on_sc=False
        ),
    )
    def k(tab_hbm, notes_hbm, on_hbm, dur_hbm, out_hbm,
          tab_v, notes_v, p_v, in_sem, out_sem):
        wid = lax.axis_index("s") * 2 + lax.axis_index("c")
        tile0 = wid * n_chunks
        pltpu.sync_copy(tab_hbm, tab_v)
        in_h = [None] * NBUF
        out_h = [None] * NBUF

        def fire_in(g):
            b = g % NBUF
            ti = tile0 + g
            t = ti // 32
            j = lax.rem(ti, 32)
            base = pl.multiple_of(ti * C, C)
            in_h[b] = [
                pltpu.async_copy(notes_hbm.at[pl.ds(base, C)], notes_v[b], in_sem[b]),
                pltpu.async_copy(on_hbm.at[t, :, j, :], p_v[b].at[EMB], in_sem[b]),
                pltpu.async_copy(dur_hbm.at[t, :, j, :], p_v[b].at[EMB + 1], in_sem[b]),
            ]

        fire_in(0)
        fire_in(1)
        for g in range(n_chunks):
            b = g % NBUF
            for h in in_h[b]:
                h.wait()

            nv = notes_v[b]
            pv = p_v[b]

            @plsc.parallel_loop(0, C, step=LANES, unroll=4)
            def gat_body(i):
                off = pl.multiple_of(i, LANES)
                r = off // 128
                cc = pl.multiple_of(lax.rem(off, 128), LANES)
                nt = nv[pl.ds(off, LANES)]
                for d in range(EMB):
                    e = plsc.load_gather(tab_v, [nt + (NUM_NOTES * d)])
                    pv[d, r, pl.ds(cc, LANES)] = e

            ti = tile0 + g
            out_h[b] = pltpu.async_copy(pv, out_hbm.at[:, ti, :, :], out_sem[b])
            if g + 2 < n_chunks:
                if g >= 1:
                    out_h[(g + 2) % NBUF].wait()
                fire_in(g + 2)
        for g in (n_chunks - 3, n_chunks - 2, n_chunks - 1):
            out_h[g % NBUF].wait()

    return k


def _tile_order(x, L, B):
    return x.T.reshape(L // 8, 8, B // 128, 128).transpose(0, 2, 1, 3).reshape(L * B)


@jax.jit
def kernel(notes, onsets, durations, note_embedding_weight):
    B, L = notes.shape
    N = B * L
    tab_cm = note_embedding_weight.T.reshape(NUM_NOTES * EMB)
    notes_p = _tile_order(notes, L, B)
    on_p = onsets[:, :, 0].T.reshape(L // 8, 8, B // 128, 128)
    dur_p = durations[:, :, 0].T.reshape(L // 8, 8, B // 128, 128)
    out = _build(N)(tab_cm, notes_p, on_p, dur_p)
    out5 = out.reshape(OUT_D, L // 8, B // 128, 8, 128)
    return out5.transpose(2, 4, 1, 3, 0).reshape(B, L, OUT_D)
